# Initial kernel scaffold; baseline (speedup 1.0000x reference)
import jax
import jax.numpy as jnp
from jax.experimental import pallas as pl


def kernel(pillar_features, coords):
    raise NotImplementedError("write your pallas kernel here")


# speedup vs baseline: 4.1287x; 4.1287x over previous
import functools

import jax
import jax.numpy as jnp
from jax import lax
from jax.experimental import pallas as pl
from jax.experimental.pallas import tpu as pltpu
from jax.experimental.pallas import tpu_sc as plsc

H, W = 512, 512
B, P, C = 4, 12000, 64
HW = H * W
NC, NS, L = 2, 16, 16
NW = NC * NS
SENT = P
PPAD = 12016
CELLS_A = HW // NW
NVREG_P = P // L
CPW = 4
NCH_GRP = C // CPW
NCELL_GRP = NW // NCH_GRP
CELLS_B = HW // NCELL_GRP
CHUNK = 8192
NCHUNK = CELLS_B // CHUNK

_MESH = plsc.VectorSubcoreMesh(
    core_axis_name="c", subcore_axis_name="s", num_cores=NC, num_subcores=NS
)


@functools.partial(
    pl.kernel,
    out_type=jax.ShapeDtypeStruct((B * HW,), jnp.int32),
    mesh=_MESH,
    scratch_types=[
        pltpu.VMEM((P,), jnp.int32),
        pltpu.VMEM((P,), jnp.int32),
        pltpu.VMEM((CELLS_A,), jnp.int32),
    ],
    compiler_params=pltpu.CompilerParams(needs_layout_passes=False),
)
def _stage_a(y_hbm, x_hbm, idximg_hbm, y_v, x_v, img_v):
    cid = lax.axis_index("c")
    sid = lax.axis_index("s")
    wid = sid * NC + cid
    base = wid * CELLS_A
    lane = lax.iota(jnp.int32, L)
    nxt = jnp.minimum(lane + 1, L - 1)
    sentv = jnp.full((L,), SENT, jnp.int32)

    for b in range(B):
        pltpu.sync_copy(y_hbm.at[pl.ds(b * P, P)], y_v)
        pltpu.sync_copy(x_hbm.at[pl.ds(b * P, P)], x_v)

        def init_body(i, carry):
            img_v[pl.ds(i * L, L)] = sentv
            return carry

        lax.fori_loop(0, CELLS_A // L, init_body, 0)

        def body(i, carry):
            yv = y_v[pl.ds(i * L, L)]
            xv = x_v[pl.ds(i * L, L)]
            ok = (xv >= 0) & (xv < W) & (yv >= 0) & (yv < H)
            linv = jnp.where(ok, yv * W + xv, -1)
            key = linv * L + lane
            sk, sv = plsc.sort_key_val(key, lane)
            lin_s = lax.shift_right_arithmetic(sk, 4)
            lin_n = lin_s.at[nxt].get(mode="promise_in_bounds")
            win = (lin_s != lin_n) | (lane == L - 1)
            inr = (lin_s >= base) & (lin_s < base + CELLS_A)
            pval = sv + i * L
            plsc.store_scatter(img_v, [lin_s - base], pval, mask=win & inr)
            return carry

        lax.fori_loop(0, NVREG_P, body, 0)
        pltpu.sync_copy(img_v, idximg_hbm.at[pl.ds(b * HW + base, CELLS_A)])


@functools.partial(
    pl.kernel,
    out_type=jax.ShapeDtypeStruct((B * C * HW,), jnp.float32),
    mesh=_MESH,
    scratch_types=[
        pltpu.VMEM((CPW * PPAD,), jnp.float32),
        pltpu.VMEM((CHUNK,), jnp.int32),
        pltpu.VMEM((CPW, CHUNK), jnp.float32),
    ],
    compiler_params=pltpu.CompilerParams(needs_layout_passes=False),
)
def _stage_b(idximg_hbm, featT_hbm, out_hbm, tab_v, idx_v, ob_v):
    cid = lax.axis_index("c")
    sid = lax.axis_index("s")
    wid = sid * NC + cid
    chg = wid % NCH_GRP
    cg = wid // NCH_GRP
    c0 = chg * CPW
    cellbase = cg * CELLS_B

    for b in range(B):
        pltpu.sync_copy(
            featT_hbm.at[pl.ds((b * C + c0) * PPAD, CPW * PPAD)], tab_v
        )
        for k in range(NCHUNK):
            off = cellbase + k * CHUNK
            pltpu.sync_copy(idximg_hbm.at[pl.ds(b * HW + off, CHUNK)], idx_v)

            def gbody(i, carry):
                vidx = idx_v[pl.ds(i * L, L)]
                for c in range(CPW):
                    ob_v[c, pl.ds(i * L, L)] = plsc.load_gather(
                        tab_v, [vidx + c * PPAD]
                    )
                return carry

            lax.fori_loop(0, CHUNK // L, gbody, 0)
            for c in range(CPW):
                pltpu.sync_copy(
                    ob_v.at[c],
                    out_hbm.at[pl.ds((b * C + c0 + c) * HW + off, CHUNK)],
                )


def kernel(pillar_features, coords):
    y = coords[:, :, 1].astype(jnp.int32).reshape(B * P)
    x = coords[:, :, 2].astype(jnp.int32).reshape(B * P)
    idximg = _stage_a(y, x)
    featT = (
        jnp.zeros((B, C, PPAD), jnp.float32)
        .at[:, :, :P]
        .set(pillar_features.transpose(0, 2, 1))
        .reshape(B * C * PPAD)
    )
    out = _stage_b(idximg, featT)
    return out.reshape(B, C, H, W)


# speedup vs baseline: 10.0494x; 2.4340x over previous
import functools

import jax
import jax.numpy as jnp
from jax import lax
from jax.experimental import pallas as pl
from jax.experimental.pallas import tpu as pltpu
from jax.experimental.pallas import tpu_sc as plsc

H, W = 512, 512
B, P, C = 4, 12000, 64
HW = H * W
NC, NS, L = 2, 16, 16
NW = NC * NS
SENT = P
PPAD = 12016
CELLS_A = HW // NW
NVREG_P = P // L
CPW = 4
NCH_GRP = C // CPW
NCELL_GRP = NW // NCH_GRP
CELLS_B = HW // NCELL_GRP
CHUNK = 8192
NCHUNK = CELLS_B // CHUNK

_MESH = plsc.VectorSubcoreMesh(
    core_axis_name="c", subcore_axis_name="s", num_cores=NC, num_subcores=NS
)


@functools.partial(
    pl.kernel,
    out_type=jax.ShapeDtypeStruct((B * HW,), jnp.int32),
    mesh=_MESH,
    scratch_types=[
        pltpu.VMEM((P,), jnp.int32),
        pltpu.VMEM((P,), jnp.int32),
        pltpu.VMEM((CELLS_A,), jnp.int32),
    ],
    compiler_params=pltpu.CompilerParams(needs_layout_passes=False),
)
def _stage_a(y_hbm, x_hbm, idximg_hbm, y_v, x_v, img_v):
    cid = lax.axis_index("c")
    sid = lax.axis_index("s")
    wid = sid * NC + cid
    base = wid * CELLS_A
    lane = lax.iota(jnp.int32, L)
    nxt = jnp.minimum(lane + 1, L - 1)
    sentv = jnp.full((L,), SENT, jnp.int32)

    for b in range(B):
        pltpu.sync_copy(y_hbm.at[pl.ds(b * P, P)], y_v)
        pltpu.sync_copy(x_hbm.at[pl.ds(b * P, P)], x_v)

        def init_body(i, carry):
            img_v[pl.ds(i * L, L)] = sentv
            return carry

        lax.fori_loop(0, CELLS_A // L, init_body, 0)

        def body(i, carry):
            yv = y_v[pl.ds(i * L, L)]
            xv = x_v[pl.ds(i * L, L)]
            ok = (xv >= 0) & (xv < W) & (yv >= 0) & (yv < H)
            linv = jnp.where(ok, yv * W + xv, -1)
            key = linv * L + lane
            sk, sv = plsc.sort_key_val(key, lane)
            lin_s = lax.shift_right_arithmetic(sk, 4)
            lin_n = lin_s.at[nxt].get(mode="promise_in_bounds")
            win = (lin_s != lin_n) | (lane == L - 1)
            inr = (lin_s >= base) & (lin_s < base + CELLS_A)
            pval = sv + i * L
            plsc.store_scatter(img_v, [lin_s - base], pval, mask=win & inr)
            return carry

        lax.fori_loop(0, NVREG_P, body, 0)
        pltpu.sync_copy(img_v, idximg_hbm.at[pl.ds(b * HW + base, CELLS_A)])


NSTEP = B * NCHUNK


@functools.partial(
    pl.kernel,
    out_type=jax.ShapeDtypeStruct((B * C * HW,), jnp.float32),
    mesh=_MESH,
    scratch_types=[
        pltpu.VMEM((CPW * PPAD,), jnp.float32),
        pltpu.VMEM((2 * CHUNK,), jnp.int32),
        pltpu.VMEM((2 * CPW * CHUNK,), jnp.float32),
        pltpu.SemaphoreType.DMA((2,)),
        pltpu.SemaphoreType.DMA((2,)),
    ],
    compiler_params=pltpu.CompilerParams(needs_layout_passes=False),
)
def _stage_b(idximg_hbm, featT_hbm, out_hbm, tab_v, idx_v, ob_v, isem, osem):
    cid = lax.axis_index("c")
    sid = lax.axis_index("s")
    wid = sid * NC + cid
    chg = wid % NCH_GRP
    cg = wid // NCH_GRP
    c0 = chg * CPW
    cellbase = cg * CELLS_B

    def idx_src(t):
        b, k = t >> 4, t & (NCHUNK - 1)
        return idximg_hbm.at[pl.ds(b * HW + cellbase + k * CHUNK, CHUNK)]

    pltpu.async_copy(idx_src(jnp.int32(0)), idx_v.at[pl.ds(0, CHUNK)],
                     isem.at[0])

    def step(t, carry):
        slot = t & 1
        b, k = t >> 4, t & (NCHUNK - 1)
        islot = pl.ds(slot * CHUNK, CHUNK)
        oslot = pl.ds(slot * CPW * CHUNK, CPW * CHUNK)

        @pl.when(t < NSTEP - 1)
        def _():
            nslot = (t + 1) & 1
            pltpu.async_copy(idx_src(t + 1),
                             idx_v.at[pl.ds(nslot * CHUNK, CHUNK)],
                             isem.at[nslot])

        @pl.when(k == 0)
        def _():
            pltpu.sync_copy(
                featT_hbm.at[pl.ds((b * C + c0) * PPAD, CPW * PPAD)], tab_v
            )

        pltpu.make_async_copy(idx_src(t), idx_v.at[islot], isem.at[slot]).wait()
        @pl.when(t >= 2)
        def _():
            pltpu.make_async_copy(out_hbm.at[pl.ds(0, CPW * CHUNK)],
                                  ob_v.at[oslot], osem.at[slot]).wait()

        @plsc.parallel_loop(0, CHUNK // L, unroll=4)
        def _(i):
            vidx = idx_v[pl.ds(slot * CHUNK + i * L, L)]
            for c in range(CPW):
                ob_v[pl.ds(slot * CPW * CHUNK + c * CHUNK + i * L, L)] = (
                    plsc.load_gather(tab_v, [vidx + c * PPAD])
                )

        off = cellbase + k * CHUNK
        for c in range(CPW):
            pltpu.async_copy(
                ob_v.at[pl.ds(slot * CPW * CHUNK + c * CHUNK, CHUNK)],
                out_hbm.at[pl.ds((b * C + c0 + c) * HW + off, CHUNK)],
                osem.at[slot],
            )
        return carry

    lax.fori_loop(0, NSTEP, step, 0)

    for slot in range(2):
        pltpu.make_async_copy(
            out_hbm.at[pl.ds(0, CPW * CHUNK)],
            ob_v.at[pl.ds(slot * CPW * CHUNK, CPW * CHUNK)],
            osem.at[slot],
        ).wait()


def kernel(pillar_features, coords):
    y = coords[:, :, 1].astype(jnp.int32).reshape(B * P)
    x = coords[:, :, 2].astype(jnp.int32).reshape(B * P)
    idximg = _stage_a(y, x)
    featT = (
        jnp.zeros((B, C, PPAD), jnp.float32)
        .at[:, :, :P]
        .set(pillar_features.transpose(0, 2, 1))
        .reshape(B * C * PPAD)
    )
    out = _stage_b(idximg, featT)
    return out.reshape(B, C, H, W)


# speedup vs baseline: 10.8980x; 1.0844x over previous
import functools

import jax
import jax.numpy as jnp
from jax import lax
from jax.experimental import pallas as pl
from jax.experimental.pallas import tpu as pltpu
from jax.experimental.pallas import tpu_sc as plsc

H, W = 512, 512
B, P, C = 4, 12000, 64
HW = H * W
NC, NS, L = 2, 16, 16
SENT = P
PPAD = 12016
CELLS_SC = HW // NC
CELLS_A = CELLS_SC // NS
NVREG_P = P // L
CPW = 4
CHUNK = 2048
NCHUNK = CELLS_SC // CHUNK

_MESH = plsc.VectorSubcoreMesh(
    core_axis_name="c", subcore_axis_name="s", num_cores=NC, num_subcores=NS
)


@functools.partial(
    pl.kernel,
    out_type=jax.ShapeDtypeStruct((B * C * HW,), jnp.float32),
    mesh=_MESH,
    scratch_types=[
        pltpu.VMEM((P,), jnp.int32),
        pltpu.VMEM((P,), jnp.int32),
        pltpu.VMEM((CELLS_A,), jnp.int32),
        pltpu.VMEM((PPAD,), jnp.uint32),
        pltpu.VMEM((CPW * PPAD,), jnp.float32),
        pltpu.VMEM((2 * CHUNK,), jnp.int32),
        pltpu.VMEM((2 * CPW * CHUNK,), jnp.float32),
        pltpu.VMEM_SHARED((2 * CELLS_SC,), jnp.int32),
        pltpu.SemaphoreType.DMA((2,)),
        pltpu.SemaphoreType.DMA((2,)),
        pltpu.SemaphoreType.DMA,
        pltpu.SemaphoreType.DMA,
    ],
    compiler_params=pltpu.CompilerParams(needs_layout_passes=False),
)
def _scatter_image(y_hbm, x_hbm, featT_hbm, out_hbm,
                   y_v, x_v, img_v, pk_v, tab_v, idx_v, ob_v, sp,
                   isem, osem, tsem, ysem):
    cid = lax.axis_index("c")
    sid = lax.axis_index("s")
    base = cid * CELLS_SC + sid * CELLS_A
    c0 = sid * CPW
    cellbase = cid * CELLS_SC
    lane = lax.iota(jnp.int32, L)
    nxt = jnp.minimum(lane + 1, L - 1)
    sentv = jnp.full((L,), SENT, jnp.int32)

    pltpu.sync_copy(y_hbm.at[pl.ds(0, P)], y_v)
    pltpu.sync_copy(x_hbm.at[pl.ds(0, P)], x_v)

    for b in range(B):
        if b > 0:
            pltpu.make_async_copy(y_hbm.at[pl.ds(b * P, P)], y_v, ysem).wait()
            pltpu.make_async_copy(x_hbm.at[pl.ds(b * P, P)], x_v, ysem).wait()
        pltpu.async_copy(
            featT_hbm.at[pl.ds((b * C + c0) * PPAD, CPW * PPAD)], tab_v, tsem
        )

        @plsc.parallel_loop(0, CELLS_A // L)
        def _(i):
            img_v[pl.ds(i * L, L)] = sentv

        def p1(i, off):
            yv = y_v[pl.ds(i * L, L)]
            xv = x_v[pl.ds(i * L, L)]
            ok = (xv >= 0) & (xv < W) & (yv >= 0) & (yv < H)
            linv = jnp.where(ok, yv * W + xv, -1)
            hit = (linv >= base) & (linv < base + CELLS_A)
            packed = (linv * 16384 + (lane + i * L)).astype(jnp.uint32)
            plsc.store_compressed(pk_v.at[pl.ds(off, L)], packed, mask=hit)
            return off + jnp.sum(hit.astype(jnp.int32))

        nhit = lax.fori_loop(0, NVREG_P, p1, jnp.int32(0))

        def p2(j, carry):
            pk = pk_v[pl.ds(j * L, L)]
            valid = (j * L + lane) < nhit
            sk, sv, sm = plsc.sort_key_val(pk, pk, mask=valid)
            lin_s = jnp.where(
                sm, lax.shift_right_logical(sk, jnp.uint32(14)).astype(jnp.int32), -7
            )
            lin_n = lin_s.at[nxt].get(mode="promise_in_bounds")
            win = (lin_s != lin_n) | (lane == L - 1)
            pval = (sv & jnp.uint32(16383)).astype(jnp.int32)
            plsc.store_scatter(img_v, [lin_s - base], pval, mask=win & sm)
            return carry

        lax.fori_loop(0, (nhit + L - 1) >> 4, p2, 0)

        pltpu.sync_copy(
            img_v,
            sp.at[pl.ds((b & 1) * CELLS_SC + sid * CELLS_A, CELLS_A)],
        )
        plsc.subcore_barrier()

        if b + 1 < B:
            pltpu.async_copy(y_hbm.at[pl.ds((b + 1) * P, P)], y_v, ysem)
            pltpu.async_copy(x_hbm.at[pl.ds((b + 1) * P, P)], x_v, ysem)
        pltpu.make_async_copy(
            featT_hbm.at[pl.ds(0, CPW * PPAD)], tab_v, tsem
        ).wait()

        spbase = (b & 1) * CELLS_SC
        pltpu.async_copy(
            sp.at[pl.ds(spbase, CHUNK)], idx_v.at[pl.ds(0, CHUNK)], isem.at[0]
        )

        def step(t, carry):
            slot = t & 1

            @pl.when(t < NCHUNK - 1)
            def _():
                ns = (t + 1) & 1
                pltpu.async_copy(
                    sp.at[pl.ds(spbase + (t + 1) * CHUNK, CHUNK)],
                    idx_v.at[pl.ds(ns * CHUNK, CHUNK)],
                    isem.at[ns],
                )

            pltpu.make_async_copy(
                sp.at[pl.ds(spbase, CHUNK)],
                idx_v.at[pl.ds(slot * CHUNK, CHUNK)],
                isem.at[slot],
            ).wait()

            @pl.when(t >= 2)
            def _():
                pltpu.make_async_copy(
                    out_hbm.at[pl.ds(0, CPW * CHUNK)],
                    ob_v.at[pl.ds(slot * CPW * CHUNK, CPW * CHUNK)],
                    osem.at[slot],
                ).wait()

            @plsc.parallel_loop(0, CHUNK // L, unroll=4)
            def _(i):
                vidx = idx_v[pl.ds(slot * CHUNK + i * L, L)]
                for c in range(CPW):
                    ob_v[pl.ds(slot * CPW * CHUNK + c * CHUNK + i * L, L)] = (
                        plsc.load_gather(tab_v, [vidx + c * PPAD])
                    )

            for c in range(CPW):
                pltpu.async_copy(
                    ob_v.at[pl.ds(slot * CPW * CHUNK + c * CHUNK, CHUNK)],
                    out_hbm.at[
                        pl.ds((b * C + c0 + c) * HW + cellbase + t * CHUNK,
                              CHUNK)
                    ],
                    osem.at[slot],
                )
            return carry

        lax.fori_loop(0, NCHUNK, step, 0)

        for slot in range(2):
            pltpu.make_async_copy(
                out_hbm.at[pl.ds(0, CPW * CHUNK)],
                ob_v.at[pl.ds(slot * CPW * CHUNK, CPW * CHUNK)],
                osem.at[slot],
            ).wait()


def kernel(pillar_features, coords):
    y = coords[:, :, 1].astype(jnp.int32).reshape(B * P)
    x = coords[:, :, 2].astype(jnp.int32).reshape(B * P)
    featT = (
        jnp.zeros((B, C, PPAD), jnp.float32)
        .at[:, :, :P]
        .set(pillar_features.transpose(0, 2, 1))
        .reshape(B * C * PPAD)
    )
    out = _scatter_image(y, x, featT)
    return out.reshape(B, C, H, W)
